# baseline (speedup 1.0000x reference)
import jax
import jax.numpy as jnp
from jax.experimental import pallas as pl
from jax.experimental.pallas import tpu as pltpu

_M = 16
_K = 3
_D = 512
_S = 2048
_C = 7
_B = 16
_N = _S - _M + 1
_EXCL = _M // 4
_R = 512
_T = _S // _R
_SUB = 128
_NSUB = _S // _SUB
_TS_PAD = _S + 128
_IMAX = 2**31 - 1
_NEG = -1e30


def _row_col(dotm, r_local, v):
    nrows = dotm.shape[0]
    oh = (jax.lax.broadcasted_iota(jnp.int32, (1, nrows), 1)
          == r_local).astype(jnp.float32)
    rowvec = jax.lax.dot_general(
        oh, dotm, (((1,), (0,)), ((), ())),
        precision=jax.lax.Precision.HIGHEST,
        preferred_element_type=jnp.float32)
    col_iota = jax.lax.broadcasted_iota(jnp.int32, (1, _S), 1)
    return jnp.min(jnp.where(rowvec == v, col_iota, _IMAX))


def _sub_top1(rowmax, row_iota, base):
    v = jnp.max(rowmax)
    r = jnp.min(jnp.where(rowmax == v, row_iota, _IMAX))
    return v, base + r


def _encembed_kernel(ts_ref, x_ref, w_ref, b_ref, out_ref, wzb_ref,
                     mk_ref, cv_ref, ci_ref):
    bb = pl.program_id(0)
    t = pl.program_id(1)

    @pl.when((bb == 0) & (t < _T))
    def _build_mask():
        row = t * _R + jax.lax.broadcasted_iota(jnp.int32, (_R, _S), 0)
        col = jax.lax.broadcasted_iota(jnp.int32, (_R, _S), 1)
        invalid = ((jnp.abs(row - col) <= _EXCL)
                   | (col >= _N) | (row >= _N))
        mk_ref[pl.ds(t * _R, _R), :] = jnp.where(invalid, _NEG, 0.0)

    @pl.when(t == 0)
    def _build_wz():
        rows = [ts_ref[0, 0:1, pl.ds(k, _S)] for k in range(_M)]
        wT = jnp.concatenate(rows, axis=0)
        mu = jnp.mean(wT, axis=0, keepdims=True)
        sd = jnp.sqrt(jnp.mean((wT - mu) ** 2, axis=0, keepdims=True)) + 1e-8
        wzb_ref[...] = ((wT - mu) / sd).astype(jnp.bfloat16)
        for i in range(2):
            cv_ref[_NSUB + i] = _NEG
            ci_ref[_NSUB + i] = _IMAX

    @pl.when(t < _T)
    def _tile_scan():
        a = wzb_ref[:, pl.ds(t * _R, _R)]
        dot = jax.lax.dot_general(
            a, wzb_ref[...], (((0,), (0,)), ((), ())),
            preferred_element_type=jnp.float32)
        dotm = dot + mk_ref[pl.ds(t * _R, _R), :]
        rowmax = jnp.max(dotm, axis=1, keepdims=True)
        row_iota = jax.lax.broadcasted_iota(jnp.int32, (_SUB, 1), 0)
        for k in range(_R // _SUB):
            v, r = _sub_top1(rowmax[k * _SUB:(k + 1) * _SUB], row_iota,
                             k * _SUB)
            c = _row_col(dotm, r, v)
            slot = t * (_R // _SUB) + k
            cv_ref[slot] = v
            ci_ref[slot] = (t * _R + r) * _S + c

    @pl.when(t == _T)
    def _finalize():
        def best(excluded):
            bv, bi = jnp.float32(_NEG), jnp.int32(_IMAX)
            for i in range(_NSUB + 2):
                cand_v, cand_i = cv_ref[i], ci_ref[i]
                ok = True
                for e in excluded:
                    ok = ok & (cand_i != e)
                better = ok & ((cand_v > bv) | ((cand_v == bv) & (cand_i < bi)))
                bv = jnp.where(better, cand_v, bv)
                bi = jnp.where(better, cand_i, bi)
            return bv, bi

        chosen = []
        for s in range(_K):
            _, ci = best(chosen)
            chosen.append(ci)
            if s < _K - 1:
                rb = (ci // (_SUB * _S)) * _SUB
                a = wzb_ref[:, pl.ds(rb, _SUB)]
                dot = jax.lax.dot_general(
                    a, wzb_ref[...], (((0,), (0,)), ((), ())),
                    preferred_element_type=jnp.float32)
                dotm = dot + mk_ref[pl.ds(rb, _SUB), :]
                row_iota2 = jax.lax.broadcasted_iota(jnp.int32, (_SUB, _S), 0)
                col_iota2 = jax.lax.broadcasted_iota(jnp.int32, (_SUB, _S), 1)
                for mf in chosen:
                    hit = ((row_iota2 == (mf >> 11) - rb)
                           & (col_iota2 == (mf & (_S - 1))))
                    dotm = jnp.where(hit, _NEG, dotm)
                rowmax = jnp.max(dotm, axis=1, keepdims=True)
                v, r = _sub_top1(
                    rowmax,
                    jax.lax.broadcasted_iota(jnp.int32, (_SUB, 1), 0), 0)
                c = _row_col(dotm, r, v)
                cv_ref[_NSUB + s] = v
                ci_ref[_NSUB + s] = (rb + r) * _S + c

        w16 = w_ref[...].astype(jnp.bfloat16)
        for s in range(_K):
            j = chosen[s] & (_S - 1)
            start = jnp.clip(j - _M // 2, 0, _S - _M)
            patch = x_ref[0, pl.ds(start, _M), :].astype(jnp.bfloat16)
            emb = jax.lax.dot_general(
                patch, w16, (((0,), (1,)), ((), ())),
                preferred_element_type=jnp.float32)
            out_ref[0, :, s, :] = emb + b_ref[...]


def kernel(x, W, b):
    ts = jnp.pad(x[:, :, 0], ((0, 0), (0, _TS_PAD - _S))).reshape(_B, 1, _TS_PAD)
    b2 = b.reshape(1, _D)
    return pl.pallas_call(
        _encembed_kernel,
        grid=(_B, _T + 1),
        in_specs=[
            pl.BlockSpec((1, 1, _TS_PAD), lambda bb, tt: (bb, 0, 0)),
            pl.BlockSpec((1, _S, _C), lambda bb, tt: (bb, 0, 0)),
            pl.BlockSpec((_D, _M), lambda bb, tt: (0, 0)),
            pl.BlockSpec((1, _D), lambda bb, tt: (0, 0)),
        ],
        out_specs=pl.BlockSpec((1, _C, _K, _D), lambda bb, tt: (bb, 0, 0, 0)),
        out_shape=jax.ShapeDtypeStruct((_B, _C, _K, _D), jnp.float32),
        scratch_shapes=[
            pltpu.VMEM((_M, _S), jnp.bfloat16),
            pltpu.VMEM((_S, _S), jnp.float32),
            pltpu.SMEM((_NSUB + 2,), jnp.float32),
            pltpu.SMEM((_NSUB + 2,), jnp.int32),
        ],
    )(ts, x, W, b2)


# speedup vs baseline: 3.6811x; 3.6811x over previous
import jax
import jax.numpy as jnp
from jax.experimental import pallas as pl
from jax.experimental.pallas import tpu as pltpu

_M = 16
_K = 3
_D = 512
_S = 2048
_C = 7
_B = 16
_N = _S - _M + 1
_EXCL = _M // 4
_R = 512
_T = _S // _R
_SUB = 128
_NSUB = _S // _SUB
_TS_PAD = _S + 128
_IMAX = 2**31 - 1
_NEG = -1e30


def _row_col(dm_ref, r_local, v):
    rowvec = dm_ref[pl.ds(r_local, 1), :]
    col_iota = jax.lax.broadcasted_iota(jnp.int32, (1, _S), 1)
    return jnp.min(jnp.where(rowvec == v, col_iota, _IMAX))


def _sub_top1(rowmax, row_iota, base):
    v = jnp.max(rowmax)
    r = jnp.min(jnp.where(rowmax == v, row_iota, _IMAX))
    return v, base + r


def _encembed_kernel(ts_ref, x_ref, w_ref, b_ref, out_ref, wzb_ref,
                     mk_ref, dm_ref, cv_ref, ci_ref):
    bb = pl.program_id(0)
    t = pl.program_id(1)

    @pl.when((bb == 0) & (t < _T))
    def _build_mask():
        row = t * _R + jax.lax.broadcasted_iota(jnp.int32, (_R, _S), 0)
        col = jax.lax.broadcasted_iota(jnp.int32, (_R, _S), 1)
        invalid = ((jnp.abs(row - col) <= _EXCL)
                   | (col >= _N) | (row >= _N))
        mk_ref[pl.ds(t * _R, _R), :] = jnp.where(invalid, _NEG, 0.0)

    @pl.when(t == 0)
    def _build_wz():
        rows = [ts_ref[0, 0:1, pl.ds(k, _S)] for k in range(_M)]
        wT = jnp.concatenate(rows, axis=0)
        mu = jnp.mean(wT, axis=0, keepdims=True)
        sd = jnp.sqrt(jnp.mean((wT - mu) ** 2, axis=0, keepdims=True)) + 1e-8
        wzb_ref[...] = ((wT - mu) / sd).astype(jnp.bfloat16)
        for i in range(2):
            cv_ref[_NSUB + i] = _NEG
            ci_ref[_NSUB + i] = _IMAX

    @pl.when(t < _T)
    def _tile_scan():
        a = wzb_ref[:, pl.ds(t * _R, _R)]
        dot = jax.lax.dot_general(
            a, wzb_ref[...], (((0,), (0,)), ((), ())),
            preferred_element_type=jnp.float32)
        dotm = dot + mk_ref[pl.ds(t * _R, _R), :]
        dm_ref[...] = dotm
        rowmax = jnp.max(dotm, axis=1, keepdims=True)
        row_iota = jax.lax.broadcasted_iota(jnp.int32, (_SUB, 1), 0)
        for k in range(_R // _SUB):
            v, r = _sub_top1(rowmax[k * _SUB:(k + 1) * _SUB], row_iota,
                             k * _SUB)
            c = _row_col(dm_ref, r, v)
            slot = t * (_R // _SUB) + k
            cv_ref[slot] = v
            ci_ref[slot] = (t * _R + r) * _S + c

    @pl.when(t == _T)
    def _finalize():
        def best(excluded):
            bv, bi = jnp.float32(_NEG), jnp.int32(_IMAX)
            for i in range(_NSUB + 2):
                cand_v, cand_i = cv_ref[i], ci_ref[i]
                ok = True
                for e in excluded:
                    ok = ok & (cand_i != e)
                better = ok & ((cand_v > bv) | ((cand_v == bv) & (cand_i < bi)))
                bv = jnp.where(better, cand_v, bv)
                bi = jnp.where(better, cand_i, bi)
            return bv, bi

        chosen = []
        for s in range(_K):
            _, ci = best(chosen)
            chosen.append(ci)
            if s < _K - 1:
                rb = (ci // (_SUB * _S)) * _SUB
                a = wzb_ref[:, pl.ds(rb, _SUB)]
                dot = jax.lax.dot_general(
                    a, wzb_ref[...], (((0,), (0,)), ((), ())),
                    preferred_element_type=jnp.float32)
                dotm = dot + mk_ref[pl.ds(rb, _SUB), :]
                row_iota2 = jax.lax.broadcasted_iota(jnp.int32, (_SUB, _S), 0)
                col_iota2 = jax.lax.broadcasted_iota(jnp.int32, (_SUB, _S), 1)
                for mf in chosen:
                    hit = ((row_iota2 == (mf >> 11) - rb)
                           & (col_iota2 == (mf & (_S - 1))))
                    dotm = jnp.where(hit, _NEG, dotm)
                dm_ref[pl.ds(0, _SUB), :] = dotm
                rowmax = jnp.max(dotm, axis=1, keepdims=True)
                v, r = _sub_top1(
                    rowmax,
                    jax.lax.broadcasted_iota(jnp.int32, (_SUB, 1), 0), 0)
                c = _row_col(dm_ref, r, v)
                cv_ref[_NSUB + s] = v
                ci_ref[_NSUB + s] = (rb + r) * _S + c

        w16 = w_ref[...].astype(jnp.bfloat16)
        for s in range(_K):
            j = chosen[s] & (_S - 1)
            start = jnp.clip(j - _M // 2, 0, _S - _M)
            patch = x_ref[0, pl.ds(start, _M), :].astype(jnp.bfloat16)
            emb = jax.lax.dot_general(
                patch, w16, (((0,), (1,)), ((), ())),
                preferred_element_type=jnp.float32)
            out_ref[0, :, s, :] = emb + b_ref[...]


def kernel(x, W, b):
    ts = jnp.pad(x[:, :, 0], ((0, 0), (0, _TS_PAD - _S))).reshape(_B, 1, _TS_PAD)
    b2 = b.reshape(1, _D)
    return pl.pallas_call(
        _encembed_kernel,
        grid=(_B, _T + 1),
        in_specs=[
            pl.BlockSpec((1, 1, _TS_PAD), lambda bb, tt: (bb, 0, 0)),
            pl.BlockSpec((1, _S, _C), lambda bb, tt: (bb, 0, 0)),
            pl.BlockSpec((_D, _M), lambda bb, tt: (0, 0)),
            pl.BlockSpec((1, _D), lambda bb, tt: (0, 0)),
        ],
        out_specs=pl.BlockSpec((1, _C, _K, _D), lambda bb, tt: (bb, 0, 0, 0)),
        out_shape=jax.ShapeDtypeStruct((_B, _C, _K, _D), jnp.float32),
        scratch_shapes=[
            pltpu.VMEM((_M, _S), jnp.bfloat16),
            pltpu.VMEM((_S, _S), jnp.float32),
            pltpu.VMEM((_R, _S), jnp.float32),
            pltpu.SMEM((_NSUB + 2,), jnp.float32),
            pltpu.SMEM((_NSUB + 2,), jnp.int32),
        ],
    )(ts, x, W, b2)


# speedup vs baseline: 4.0614x; 1.1033x over previous
import jax
import jax.numpy as jnp
from jax.experimental import pallas as pl
from jax.experimental.pallas import tpu as pltpu

_M = 16
_K = 3
_D = 512
_S = 2048
_C = 7
_B = 16
_N = _S - _M + 1
_EXCL = _M // 4
_R = 512
_T = _S // _R
_TS_PAD = _S + 128
_IMAX = 2**31 - 1
_NEG = float("-inf")


def _tile_top1(wzb_ref, mk_ref, dm_ref, tile, row_base, masked_flats):
    a = wzb_ref[:, pl.ds(row_base, _R)]
    dot = jax.lax.dot_general(
        a, wzb_ref[...], (((0,), (0,)), ((), ())),
        preferred_element_type=jnp.float32)
    dotm = dot + mk_ref[tile, :, :]
    for mf in masked_flats:
        hit = ((jax.lax.broadcasted_iota(jnp.int32, (_R, _S), 0)
                == (mf >> 11) - row_base)
               & (jax.lax.broadcasted_iota(jnp.int32, (_R, _S), 1)
                  == (mf & (_S - 1))))
        dotm = jnp.where(hit, _NEG, dotm)
    dm_ref[...] = dotm
    rowmax = jnp.max(dotm, axis=1, keepdims=True)
    v = jnp.max(rowmax)
    row_iota = jax.lax.broadcasted_iota(jnp.int32, (_R, 1), 0)
    r = jnp.min(jnp.where(rowmax == v, row_iota, _IMAX))
    rowvec = dm_ref[pl.ds(r, 1), :]
    col_iota = jax.lax.broadcasted_iota(jnp.int32, (1, _S), 1)
    c = jnp.min(jnp.where(rowvec == v, col_iota, _IMAX))
    return v, (row_base + r) * _S + c


def _encembed_kernel(ts_ref, x_ref, w_ref, b_ref, out_ref, wzb_ref, mk_ref,
                     dm_ref, cv_ref, ci_ref):
    bb = pl.program_id(0)
    t = pl.program_id(1)

    @pl.when((bb == 0) & (t < _T))
    def _build_mask():
        row = t * _R + jax.lax.broadcasted_iota(jnp.int32, (_R, _S), 0)
        col = jax.lax.broadcasted_iota(jnp.int32, (_R, _S), 1)
        invalid = ((jnp.abs(row - col) <= _EXCL)
                   | (col >= _N) | (row >= _N))
        mk_ref[t, :, :] = jnp.where(invalid, _NEG, 0.0)

    @pl.when(t == 0)
    def _build_wz():
        rows = [ts_ref[0, 0:1, pl.ds(k, _S)] for k in range(_M)]
        wT = jnp.concatenate(rows, axis=0)
        mu = jnp.mean(wT, axis=0, keepdims=True)
        sd = jnp.sqrt(jnp.mean((wT - mu) ** 2, axis=0, keepdims=True)) + 1e-8
        wzb_ref[...] = ((wT - mu) / sd).astype(jnp.bfloat16)
        for i in range(2):
            cv_ref[_T + i] = _NEG
            ci_ref[_T + i] = _IMAX

    @pl.when(t < _T)
    def _tile_scan():
        v, f = _tile_top1(wzb_ref, mk_ref, dm_ref, t, t * _R, ())
        cv_ref[t] = v
        ci_ref[t] = f

    @pl.when(t == _T)
    def _finalize():
        def best(excluded):
            bv, bi = jnp.float32(_NEG), jnp.int32(_IMAX)
            for i in range(_T + 2):
                cand_v, cand_i = cv_ref[i], ci_ref[i]
                ok = True
                for e in excluded:
                    ok = ok & (cand_i != e)
                better = ok & ((cand_v > bv) | ((cand_v == bv) & (cand_i < bi)))
                bv = jnp.where(better, cand_v, bv)
                bi = jnp.where(better, cand_i, bi)
            return bv, bi

        chosen = []
        for s in range(_K):
            _, ci = best(chosen)
            chosen.append(ci)
            if s < _K - 1:
                tile = ci // (_R * _S)
                v, f = _tile_top1(wzb_ref, mk_ref, dm_ref, tile, tile * _R,
                                  chosen)
                cv_ref[_T + s] = v
                ci_ref[_T + s] = f

        w16 = w_ref[...].astype(jnp.bfloat16)
        for s in range(_K):
            j = chosen[s] & (_S - 1)
            start = jnp.clip(j - _M // 2, 0, _S - _M)
            patch = x_ref[0, pl.ds(start, _M), :].astype(jnp.bfloat16)
            emb = jax.lax.dot_general(
                patch, w16, (((0,), (1,)), ((), ())),
                preferred_element_type=jnp.float32)
            out_ref[0, :, s, :] = emb + b_ref[...]


def kernel(x, W, b):
    ts = jnp.pad(x[:, :, 0], ((0, 0), (0, _TS_PAD - _S))).reshape(_B, 1, _TS_PAD)
    b2 = b.reshape(1, _D)
    return pl.pallas_call(
        _encembed_kernel,
        grid=(_B, _T + 1),
        in_specs=[
            pl.BlockSpec((1, 1, _TS_PAD), lambda bb, tt: (bb, 0, 0)),
            pl.BlockSpec((1, _S, _C), lambda bb, tt: (bb, 0, 0)),
            pl.BlockSpec((_D, _M), lambda bb, tt: (0, 0)),
            pl.BlockSpec((1, _D), lambda bb, tt: (0, 0)),
        ],
        out_specs=pl.BlockSpec((1, _C, _K, _D), lambda bb, tt: (bb, 0, 0, 0)),
        out_shape=jax.ShapeDtypeStruct((_B, _C, _K, _D), jnp.float32),
        scratch_shapes=[
            pltpu.VMEM((_M, _S), jnp.bfloat16),
            pltpu.VMEM((_T, _R, _S), jnp.float32),
            pltpu.VMEM((_R, _S), jnp.float32),
            pltpu.SMEM((_T + 2,), jnp.float32),
            pltpu.SMEM((_T + 2,), jnp.int32),
        ],
    )(ts, x, W, b2)
